# baseline (speedup 1.0000x reference)
import functools

import jax
import jax.numpy as jnp
from jax import lax
from jax.experimental import pallas as pl
from jax.experimental.pallas import tpu as pltpu
from jax.experimental.pallas import tpu_sc as plsc

N_FIELDS = 26
VOCAB = 100000
EMBED = 32
BATCH = 16384
LROW = 128

NC = 2
NS = 16
NW = NC * NS

VROWS = 100008

TOT = BATCH * N_FIELDS
PER_W = TOT // NW
CHUNK = 832
NCHUNK = PER_W // CHUNK

_mesh = plsc.VectorSubcoreMesh(core_axis_name="c", subcore_axis_name="s")


@functools.partial(
    pl.kernel,
    mesh=_mesh,
    out_type=jax.ShapeDtypeStruct((TOT, EMBED), jnp.float32),
    scratch_types=[
        pltpu.VMEM((PER_W,), jnp.int32),
        pltpu.VMEM((CHUNK, LROW), jnp.float32),
        pltpu.SemaphoreType.DMA,
    ],
    compiler_params=pltpu.CompilerParams(use_tc_tiling_on_sc=False),
)
def _gather_kernel(xg_hbm, tab_hbm, out_hbm, idx_v, rows_v, sem):
    wid = lax.axis_index("s") * NC + lax.axis_index("c")
    base = wid * PER_W

    pltpu.sync_copy(xg_hbm.at[pl.ds(base, PER_W)], idx_v)

    lanes = lax.iota(jnp.int32, 16)

    def add_body(r, _):
        for j in range(8):
            s = r * 128 + j * 16
            p0 = base + s
            field = lax.rem(p0 + lanes, N_FIELDS)
            idx_v[pl.ds(s, 16)] = idx_v[pl.ds(s, 16)] + field * VROWS
        return 0

    lax.fori_loop(0, PER_W // 128, add_body, 0)

    def chunk_body(c, _):
        kb = c * CHUNK
        pltpu.async_copy(tab_hbm.at[idx_v.at[pl.ds(kb, CHUNK)]], rows_v, sem).wait()
        pltpu.sync_copy(
            rows_v.at[:, pl.ds(0, EMBED)], out_hbm.at[pl.ds(base + kb, CHUNK)]
        )
        return 0

    lax.fori_loop(0, NCHUNK, chunk_body, 0)


def kernel(x, tables):
    xg = x.reshape(-1).astype(jnp.int32)
    tabp = jnp.pad(tables, ((0, 0), (0, VROWS - VOCAB - 1), (0, LROW - EMBED)))
    tab = tabp.reshape(N_FIELDS * VROWS, LROW)
    out = _gather_kernel(xg, tab)
    return out.reshape(BATCH, N_FIELDS * EMBED)
